# baseline (speedup 1.0000x reference)
import jax
import jax.numpy as jnp
from jax.experimental import pallas as pl
from jax.experimental.pallas import tpu as pltpu

P = 512
E = 32
H = 64
BP = 8


EP = 40


def _lstm_block(corr_ref, ht_ref, ct_ref, nei_ref,
                w_emb_ref, w_ih_ref, w_hh_ref,
                ht_out_ref, ct_out_ref):
    w_emb = w_emb_ref[...]
    w_ih = w_ih_ref[...]
    w_hh = w_hh_ref[...]
    dn = (((1,), (0,)), ((), ()))

    emb_all = jnp.maximum(
        jax.lax.dot_general(w_emb, corr_ref[...], dn,
                            preferred_element_type=jnp.float32),
        0.0).astype(jnp.bfloat16)

    for p in range(BP):
        ht = ht_ref[H * p:H * p + H, :]
        ct = ct_ref[H * p:H * p + H, :]
        m = nei_ref[p:p + 1, :] > 0

        gates = (jax.lax.dot_general(w_ih, emb_all[EP * p:EP * p + EP, :],
                                     dn, preferred_element_type=jnp.float32)
                 + jax.lax.dot_general(w_hh, ht.astype(jnp.bfloat16), dn,
                                       preferred_element_type=jnp.float32))

        ti = jnp.tanh(gates[0:H, :])
        tf = jnp.tanh(gates[H:2 * H, :])
        g = jnp.tanh(gates[2 * H:3 * H, :])
        to = jnp.tanh(gates[3 * H:4 * H, :])

        c_new = 0.5 * ((ct + tf * ct) + (g + ti * g))
        tc = jnp.tanh(c_new)
        h_new = 0.5 * (tc + to * tc)

        ht_out_ref[H * p:H * p + H, :] = jnp.where(m, h_new, ht)
        ct_out_ref[H * p:H * p + H, :] = jnp.where(m, c_new, ct)


def kernel(corr_index, rela_ht, rela_ct, nei_index, W_emb, b_emb, W_ih, W_hh, b_ih, b_hh):
    ht = rela_ht.transpose(0, 2, 1).reshape(P * H, P)
    ct = rela_ct.transpose(0, 2, 1).reshape(P * H, P)
    corr_t = corr_index.transpose(0, 2, 1)
    ones_row = jnp.ones((P, 1, P), jnp.float32)
    corr_aug = jnp.concatenate([corr_t, ones_row], axis=1)
    corr_aug = corr_aug.reshape(P * 3, P).astype(jnp.bfloat16)
    nei = nei_index.astype(jnp.int32)

    w_emb_aug = jnp.concatenate([W_emb, b_emb[:, None]], axis=1)
    w_emb_aug = jnp.concatenate(
        [w_emb_aug, jnp.array([[0.0, 0.0, 1.0]], jnp.float32),
         jnp.zeros((EP - E - 1, 3), jnp.float32)], axis=0)

    w_emb_bd = jax.scipy.linalg.block_diag(
        *([w_emb_aug] * BP)).astype(jnp.bfloat16)

    bias = (b_ih + b_hh)[:, None]
    w_ih_aug = jnp.concatenate(
        [W_ih, bias, jnp.zeros((4 * H, EP - E - 1), jnp.float32)],
        axis=1)
    scale = jnp.where((jnp.arange(4 * H) < 2 * H) | (jnp.arange(4 * H) >= 3 * H),
                      0.5, 1.0)[:, None]
    w_ih_s = (w_ih_aug * scale).astype(jnp.bfloat16)
    w_hh_s = (W_hh * scale).astype(jnp.bfloat16)

    grid = (P // BP,)
    ht_out, ct_out = pl.pallas_call(
        _lstm_block,
        grid=grid,
        in_specs=[
            pl.BlockSpec((3 * BP, P), lambda i: (i, 0)),
            pl.BlockSpec((H * BP, P), lambda i: (i, 0)),
            pl.BlockSpec((H * BP, P), lambda i: (i, 0)),
            pl.BlockSpec((BP, P), lambda i: (i, 0)),
            pl.BlockSpec((BP * EP, BP * 3), lambda i: (0, 0)),
            pl.BlockSpec((4 * H, EP), lambda i: (0, 0)),
            pl.BlockSpec((4 * H, H), lambda i: (0, 0)),
        ],
        out_specs=[
            pl.BlockSpec((H * BP, P), lambda i: (i, 0)),
            pl.BlockSpec((H * BP, P), lambda i: (i, 0)),
        ],
        out_shape=[
            jax.ShapeDtypeStruct((P * H, P), jnp.float32),
            jax.ShapeDtypeStruct((P * H, P), jnp.float32),
        ],
        compiler_params=pltpu.CompilerParams(
            dimension_semantics=("arbitrary",),
        ),
    )(corr_aug, ht, ct, nei, w_emb_bd, w_ih_s, w_hh_s)

    return (ht_out.reshape(P, H, P).transpose(0, 2, 1),
            ct_out.reshape(P, H, P).transpose(0, 2, 1))


# speedup vs baseline: 1.2248x; 1.2248x over previous
import jax
import jax.numpy as jnp
from jax.experimental import pallas as pl
from jax.experimental.pallas import tpu as pltpu

P = 512
E = 32
H = 64
BP = 32


EP = 40


def _lstm_block(corr_ref, ht_ref, ct_ref, nei_ref,
                w_emb_ref, w_ih_ref, w_hh_ref,
                ht_out_ref, ct_out_ref):
    w_emb = w_emb_ref[...]
    w_ih = w_ih_ref[...]
    w_hh = w_hh_ref[...]
    dn = (((1,), (0,)), ((), ()))

    emb_all = jnp.maximum(
        jax.lax.dot_general(w_emb, corr_ref[...], dn,
                            preferred_element_type=jnp.float32),
        0.0).astype(jnp.bfloat16)

    for p in range(BP):
        ht = ht_ref[H * p:H * p + H, :]
        ct = ct_ref[H * p:H * p + H, :]
        m = nei_ref[p:p + 1, :] > 0

        gates = (jax.lax.dot_general(w_ih, emb_all[EP * p:EP * p + EP, :],
                                     dn, preferred_element_type=jnp.float32)
                 + jax.lax.dot_general(w_hh, ht.astype(jnp.bfloat16), dn,
                                       preferred_element_type=jnp.float32))

        ti = jnp.tanh(gates[0:H, :])
        tf = jnp.tanh(gates[H:2 * H, :])
        g = jnp.tanh(gates[2 * H:3 * H, :])
        to = jnp.tanh(gates[3 * H:4 * H, :])

        c_new = 0.5 * ((ct + tf * ct) + (g + ti * g))
        tc = jnp.tanh(c_new)
        h_new = 0.5 * (tc + to * tc)

        ht_out_ref[H * p:H * p + H, :] = jnp.where(m, h_new, ht)
        ct_out_ref[H * p:H * p + H, :] = jnp.where(m, c_new, ct)


def kernel(corr_index, rela_ht, rela_ct, nei_index, W_emb, b_emb, W_ih, W_hh, b_ih, b_hh):
    ht = rela_ht.transpose(0, 2, 1).reshape(P * H, P)
    ct = rela_ct.transpose(0, 2, 1).reshape(P * H, P)
    corr_t = corr_index.transpose(0, 2, 1)
    ones_row = jnp.ones((P, 1, P), jnp.float32)
    corr_aug = jnp.concatenate([corr_t, ones_row], axis=1)
    corr_aug = corr_aug.reshape(P * 3, P).astype(jnp.bfloat16)
    nei = nei_index.astype(jnp.int32)

    w_emb_aug = jnp.concatenate([W_emb, b_emb[:, None]], axis=1)
    w_emb_aug = jnp.concatenate(
        [w_emb_aug, jnp.array([[0.0, 0.0, 1.0]], jnp.float32),
         jnp.zeros((EP - E - 1, 3), jnp.float32)], axis=0)

    w_emb_bd = jax.scipy.linalg.block_diag(
        *([w_emb_aug] * BP)).astype(jnp.bfloat16)

    bias = (b_ih + b_hh)[:, None]
    w_ih_aug = jnp.concatenate(
        [W_ih, bias, jnp.zeros((4 * H, EP - E - 1), jnp.float32)],
        axis=1)
    scale = jnp.where((jnp.arange(4 * H) < 2 * H) | (jnp.arange(4 * H) >= 3 * H),
                      0.5, 1.0)[:, None]
    w_ih_s = (w_ih_aug * scale).astype(jnp.bfloat16)
    w_hh_s = (W_hh * scale).astype(jnp.bfloat16)

    grid = (P // BP,)
    ht_out, ct_out = pl.pallas_call(
        _lstm_block,
        grid=grid,
        in_specs=[
            pl.BlockSpec((3 * BP, P), lambda i: (i, 0)),
            pl.BlockSpec((H * BP, P), lambda i: (i, 0)),
            pl.BlockSpec((H * BP, P), lambda i: (i, 0)),
            pl.BlockSpec((BP, P), lambda i: (i, 0)),
            pl.BlockSpec((BP * EP, BP * 3), lambda i: (0, 0)),
            pl.BlockSpec((4 * H, EP), lambda i: (0, 0)),
            pl.BlockSpec((4 * H, H), lambda i: (0, 0)),
        ],
        out_specs=[
            pl.BlockSpec((H * BP, P), lambda i: (i, 0)),
            pl.BlockSpec((H * BP, P), lambda i: (i, 0)),
        ],
        out_shape=[
            jax.ShapeDtypeStruct((P * H, P), jnp.float32),
            jax.ShapeDtypeStruct((P * H, P), jnp.float32),
        ],
        compiler_params=pltpu.CompilerParams(
            dimension_semantics=("arbitrary",),
        ),
    )(corr_aug, ht, ct, nei, w_emb_bd, w_ih_s, w_hh_s)

    return (ht_out.reshape(P, H, P).transpose(0, 2, 1),
            ct_out.reshape(P, H, P).transpose(0, 2, 1))


# speedup vs baseline: 1.2287x; 1.0032x over previous
import jax
import jax.numpy as jnp
from jax.experimental import pallas as pl
from jax.experimental.pallas import tpu as pltpu

P = 512
E = 32
H = 64
BP = 32


EP = 40


def _lstm_block(corr_ref, ht_ref, ct_ref, nei_ref,
                w_emb_ref, w_ih_ref, w_hh_ref,
                ht_out_ref, ct_out_ref):
    w_emb = w_emb_ref[...]
    w_ih = w_ih_ref[...]
    w_hh = w_hh_ref[...]
    dn = (((1,), (0,)), ((), ()))

    emb_all = jnp.maximum(
        jax.lax.dot_general(w_emb, corr_ref[...], dn,
                            preferred_element_type=jnp.float32),
        0.0).astype(jnp.bfloat16)

    for p in range(BP):
        ht = ht_ref[H * p:H * p + H, :]
        ct = ct_ref[H * p:H * p + H, :]
        m = nei_ref[p:p + 1, :] > 0

        gates = (jax.lax.dot_general(w_ih, emb_all[EP * p:EP * p + EP, :],
                                     dn, preferred_element_type=jnp.float32)
                 + jax.lax.dot_general(w_hh, ht.astype(jnp.bfloat16), dn,
                                       preferred_element_type=jnp.float32))

        ti = jnp.tanh(gates[0:H, :])
        tf = jnp.tanh(gates[H:2 * H, :])
        g = jnp.tanh(gates[2 * H:3 * H, :])
        to = jnp.tanh(gates[3 * H:4 * H, :])

        c_new = 0.5 * ((ct + tf * ct) + (g + ti * g))
        tc = jnp.tanh(c_new)
        h_new = 0.5 * (tc + to * tc)

        ht_out_ref[H * p:H * p + H, :] = jnp.where(m, h_new, ht)
        ct_out_ref[H * p:H * p + H, :] = jnp.where(m, c_new, ct)


def kernel(corr_index, rela_ht, rela_ct, nei_index, W_emb, b_emb, W_ih, W_hh, b_ih, b_hh):
    ht = rela_ht.transpose(0, 2, 1).reshape(P * H, P)
    ct = rela_ct.transpose(0, 2, 1).reshape(P * H, P)
    corr_t = corr_index.transpose(0, 2, 1)
    ones_row = jnp.ones((P, 1, P), jnp.float32)
    corr_aug = jnp.concatenate([corr_t, ones_row], axis=1)
    corr_aug = corr_aug.reshape(P * 3, P).astype(jnp.bfloat16)
    nei = nei_index.astype(jnp.int32)

    w_emb_aug = jnp.concatenate([W_emb, b_emb[:, None]], axis=1)
    w_emb_aug = jnp.concatenate(
        [w_emb_aug, jnp.array([[0.0, 0.0, 1.0]], jnp.float32),
         jnp.zeros((EP - E - 1, 3), jnp.float32)], axis=0)

    w_emb_bd = jax.scipy.linalg.block_diag(
        *([w_emb_aug] * BP)).astype(jnp.bfloat16)

    bias = (b_ih + b_hh)[:, None]
    w_ih_aug = jnp.concatenate(
        [W_ih, bias, jnp.zeros((4 * H, EP - E - 1), jnp.float32)],
        axis=1)
    scale = jnp.where((jnp.arange(4 * H) < 2 * H) | (jnp.arange(4 * H) >= 3 * H),
                      0.5, 1.0)[:, None]
    w_ih_s = (w_ih_aug * scale).astype(jnp.bfloat16)
    w_hh_s = (W_hh * scale).astype(jnp.bfloat16)

    grid = (P // BP,)
    ht_out, ct_out = pl.pallas_call(
        _lstm_block,
        grid=grid,
        in_specs=[
            pl.BlockSpec((3 * BP, P), lambda i: (i, 0)),
            pl.BlockSpec((H * BP, P), lambda i: (i, 0)),
            pl.BlockSpec((H * BP, P), lambda i: (i, 0)),
            pl.BlockSpec((BP, P), lambda i: (i, 0)),
            pl.BlockSpec((BP * EP, BP * 3), lambda i: (0, 0)),
            pl.BlockSpec((4 * H, EP), lambda i: (0, 0)),
            pl.BlockSpec((4 * H, H), lambda i: (0, 0)),
        ],
        out_specs=[
            pl.BlockSpec((H * BP, P), lambda i: (i, 0)),
            pl.BlockSpec((H * BP, P), lambda i: (i, 0)),
        ],
        out_shape=[
            jax.ShapeDtypeStruct((P * H, P), jnp.float32),
            jax.ShapeDtypeStruct((P * H, P), jnp.float32),
        ],
        compiler_params=pltpu.CompilerParams(
            dimension_semantics=("parallel",),
        ),
    )(corr_aug, ht, ct, nei, w_emb_bd, w_ih_s, w_hh_s)

    return (ht_out.reshape(P, H, P).transpose(0, 2, 1),
            ct_out.reshape(P, H, P).transpose(0, 2, 1))


# speedup vs baseline: 1.2740x; 1.0369x over previous
import jax
import jax.numpy as jnp
from jax.experimental import pallas as pl
from jax.experimental.pallas import tpu as pltpu

P = 512
E = 32
H = 64
BP = 32


EP = 48
KS = EP + H


def _lstm_block(corr_ref, ht_ref, ct_ref, nei_ref,
                w_emb_ref, w_cat_ref,
                ht_out_ref, ct_out_ref, rhs_ref):
    w_emb = w_emb_ref[...]
    w_cat = w_cat_ref[...]
    dn = (((1,), (0,)), ((), ()))

    emb_all = jnp.maximum(
        jax.lax.dot_general(w_emb, corr_ref[...], dn,
                            preferred_element_type=jnp.float32),
        0.0).astype(jnp.bfloat16)

    for p in range(BP):
        base = KS * p
        rhs_ref[base:base + EP, :] = emb_all[EP * p:EP * p + EP, :]
        rhs_ref[base + EP:base + KS, :] = (
            ht_ref[H * p:H * p + H, :].astype(jnp.bfloat16))

    for p in range(BP):
        ht = ht_ref[H * p:H * p + H, :]
        ct = ct_ref[H * p:H * p + H, :]
        m = nei_ref[p:p + 1, :] > 0

        gates = jax.lax.dot_general(w_cat, rhs_ref[KS * p:KS * p + KS, :],
                                    dn, preferred_element_type=jnp.float32)

        ti = jnp.tanh(gates[0:H, :])
        tf = jnp.tanh(gates[H:2 * H, :])
        g = jnp.tanh(gates[2 * H:3 * H, :])
        to = jnp.tanh(gates[3 * H:4 * H, :])

        c_new = 0.5 * ((ct + tf * ct) + (g + ti * g))
        tc = jnp.tanh(c_new)
        h_new = 0.5 * (tc + to * tc)

        ht_out_ref[H * p:H * p + H, :] = jnp.where(m, h_new, ht)
        ct_out_ref[H * p:H * p + H, :] = jnp.where(m, c_new, ct)


def kernel(corr_index, rela_ht, rela_ct, nei_index, W_emb, b_emb, W_ih, W_hh, b_ih, b_hh):
    ht = rela_ht.transpose(0, 2, 1).reshape(P * H, P)
    ct = rela_ct.transpose(0, 2, 1).reshape(P * H, P)
    corr_t = corr_index.transpose(0, 2, 1)
    ones_row = jnp.ones((P, 1, P), jnp.float32)
    corr_aug = jnp.concatenate([corr_t, ones_row], axis=1)
    corr_aug = corr_aug.reshape(P * 3, P).astype(jnp.bfloat16)
    nei = nei_index.astype(jnp.int32)

    w_emb_aug = jnp.concatenate([W_emb, b_emb[:, None]], axis=1)
    w_emb_aug = jnp.concatenate(
        [w_emb_aug, jnp.array([[0.0, 0.0, 1.0]], jnp.float32),
         jnp.zeros((EP - E - 1, 3), jnp.float32)], axis=0)

    w_emb_bd = jax.scipy.linalg.block_diag(
        *([w_emb_aug] * BP)).astype(jnp.bfloat16)

    bias = (b_ih + b_hh)[:, None]
    w_ih_aug = jnp.concatenate(
        [W_ih, bias, jnp.zeros((4 * H, EP - E - 1), jnp.float32)],
        axis=1)
    scale = jnp.where((jnp.arange(4 * H) < 2 * H) | (jnp.arange(4 * H) >= 3 * H),
                      0.5, 1.0)[:, None]
    w_cat = (jnp.concatenate([w_ih_aug, W_hh], axis=1)
             * scale).astype(jnp.bfloat16)

    grid = (P // BP,)
    ht_out, ct_out = pl.pallas_call(
        _lstm_block,
        grid=grid,
        in_specs=[
            pl.BlockSpec((3 * BP, P), lambda i: (i, 0)),
            pl.BlockSpec((H * BP, P), lambda i: (i, 0)),
            pl.BlockSpec((H * BP, P), lambda i: (i, 0)),
            pl.BlockSpec((BP, P), lambda i: (i, 0)),
            pl.BlockSpec((BP * EP, BP * 3), lambda i: (0, 0)),
            pl.BlockSpec((4 * H, KS), lambda i: (0, 0)),
        ],
        out_specs=[
            pl.BlockSpec((H * BP, P), lambda i: (i, 0)),
            pl.BlockSpec((H * BP, P), lambda i: (i, 0)),
        ],
        out_shape=[
            jax.ShapeDtypeStruct((P * H, P), jnp.float32),
            jax.ShapeDtypeStruct((P * H, P), jnp.float32),
        ],
        scratch_shapes=[pltpu.VMEM((BP * KS, P), jnp.bfloat16)],
        compiler_params=pltpu.CompilerParams(
            dimension_semantics=("parallel",),
        ),
    )(corr_aug, ht, ct, nei, w_emb_bd, w_cat)

    return (ht_out.reshape(P, H, P).transpose(0, 2, 1),
            ct_out.reshape(P, H, P).transpose(0, 2, 1))


# speedup vs baseline: 1.3254x; 1.0403x over previous
import jax
import jax.numpy as jnp
from jax.experimental import pallas as pl
from jax.experimental.pallas import tpu as pltpu

P = 512
E = 32
H = 64
BP = 32


EP = 48
KS = EP + H


def _lstm_block(corr_ref, ht_ref, ct_ref, nei_ref,
                w_emb_ref, w_cat_ref,
                ht_out_ref, ct_out_ref, rhs_ref):
    w_emb = w_emb_ref[...]
    w_cat = w_cat_ref[...]
    dn = (((1,), (0,)), ((), ()))

    emb_all = jnp.maximum(
        jax.lax.dot_general(w_emb, corr_ref[...], dn,
                            preferred_element_type=jnp.float32),
        0.0).astype(jnp.bfloat16)

    for p in range(BP):
        base = KS * p
        rhs_ref[base:base + EP, :] = emb_all[EP * p:EP * p + EP, :]
        rhs_ref[base + EP:base + KS, :] = (
            ht_ref[H * p:H * p + H, :].astype(jnp.bfloat16))

    for p in range(BP):
        ht = ht_ref[H * p:H * p + H, :]
        ct = ct_ref[H * p:H * p + H, :]
        m = nei_ref[p:p + 1, :] > 0

        gates = jax.lax.dot_general(
            w_cat, rhs_ref[KS * p:KS * p + KS, :],
            dn, preferred_element_type=jnp.float32).astype(jnp.bfloat16)

        ti = jnp.tanh(gates[0:H, :])
        tf = jnp.tanh(gates[H:2 * H, :])
        g = jnp.tanh(gates[2 * H:3 * H, :])
        to = jnp.tanh(gates[3 * H:4 * H, :])

        cth = ct.astype(jnp.bfloat16)
        c_new = 0.5 * ((cth + tf * cth) + (g + ti * g))
        tc = jnp.tanh(c_new)
        h_new = 0.5 * (tc + to * tc)

        ht_out_ref[H * p:H * p + H, :] = jnp.where(
            m, h_new.astype(jnp.float32), ht)
        ct_out_ref[H * p:H * p + H, :] = jnp.where(
            m, c_new.astype(jnp.float32), ct)


def kernel(corr_index, rela_ht, rela_ct, nei_index, W_emb, b_emb, W_ih, W_hh, b_ih, b_hh):
    ht = rela_ht.transpose(0, 2, 1).reshape(P * H, P)
    ct = rela_ct.transpose(0, 2, 1).reshape(P * H, P)
    corr_t = corr_index.transpose(0, 2, 1)
    ones_row = jnp.ones((P, 1, P), jnp.float32)
    corr_aug = jnp.concatenate([corr_t, ones_row], axis=1)
    corr_aug = corr_aug.reshape(P * 3, P).astype(jnp.bfloat16)
    nei = nei_index.astype(jnp.int32)

    w_emb_aug = jnp.concatenate([W_emb, b_emb[:, None]], axis=1)
    w_emb_aug = jnp.concatenate(
        [w_emb_aug, jnp.array([[0.0, 0.0, 1.0]], jnp.float32),
         jnp.zeros((EP - E - 1, 3), jnp.float32)], axis=0)

    w_emb_bd = jax.scipy.linalg.block_diag(
        *([w_emb_aug] * BP)).astype(jnp.bfloat16)

    bias = (b_ih + b_hh)[:, None]
    w_ih_aug = jnp.concatenate(
        [W_ih, bias, jnp.zeros((4 * H, EP - E - 1), jnp.float32)],
        axis=1)
    scale = jnp.where((jnp.arange(4 * H) < 2 * H) | (jnp.arange(4 * H) >= 3 * H),
                      0.5, 1.0)[:, None]
    w_cat = (jnp.concatenate([w_ih_aug, W_hh], axis=1)
             * scale).astype(jnp.bfloat16)

    grid = (P // BP,)
    ht_out, ct_out = pl.pallas_call(
        _lstm_block,
        grid=grid,
        in_specs=[
            pl.BlockSpec((3 * BP, P), lambda i: (i, 0)),
            pl.BlockSpec((H * BP, P), lambda i: (i, 0)),
            pl.BlockSpec((H * BP, P), lambda i: (i, 0)),
            pl.BlockSpec((BP, P), lambda i: (i, 0)),
            pl.BlockSpec((BP * EP, BP * 3), lambda i: (0, 0)),
            pl.BlockSpec((4 * H, KS), lambda i: (0, 0)),
        ],
        out_specs=[
            pl.BlockSpec((H * BP, P), lambda i: (i, 0)),
            pl.BlockSpec((H * BP, P), lambda i: (i, 0)),
        ],
        out_shape=[
            jax.ShapeDtypeStruct((P * H, P), jnp.float32),
            jax.ShapeDtypeStruct((P * H, P), jnp.float32),
        ],
        scratch_shapes=[pltpu.VMEM((BP * KS, P), jnp.bfloat16)],
        compiler_params=pltpu.CompilerParams(
            dimension_semantics=("parallel",),
        ),
    )(corr_aug, ht, ct, nei, w_emb_bd, w_cat)

    return (ht_out.reshape(P, H, P).transpose(0, 2, 1),
            ct_out.reshape(P, H, P).transpose(0, 2, 1))


# speedup vs baseline: 1.3276x; 1.0017x over previous
import jax
import jax.numpy as jnp
from jax.experimental import pallas as pl
from jax.experimental.pallas import tpu as pltpu

P = 512
E = 32
H = 64
BP = 32


EP = 48
KS = EP + H


def _lstm_block(corr_ref, ht_ref, ct_ref, nei_ref,
                w_emb_ref, w_cat_ref,
                ht_out_ref, ct_out_ref, rhs_ref):
    w_emb = w_emb_ref[...]
    w_cat = w_cat_ref[...]
    dn = (((1,), (0,)), ((), ()))

    emb_all = jnp.maximum(
        jax.lax.dot_general(w_emb, corr_ref[...], dn,
                            preferred_element_type=jnp.float32),
        0.0).astype(jnp.bfloat16)

    for p in range(BP):
        base = KS * p
        rhs_ref[base:base + EP, :] = emb_all[EP * p:EP * p + EP, :]
        rhs_ref[base + EP:base + KS, :] = (
            ht_ref[H * p:H * p + H, :].astype(jnp.bfloat16))

    for p in range(BP):
        ht = ht_ref[H * p:H * p + H, :]
        ct = ct_ref[H * p:H * p + H, :]
        m = nei_ref[p:p + 1, :] > 0

        gates = jax.lax.dot_general(
            w_cat, rhs_ref[KS * p:KS * p + KS, :],
            dn, preferred_element_type=jnp.float32).astype(jnp.bfloat16)

        ti = jnp.tanh(gates[0:H, :])
        tf = jnp.tanh(gates[H:2 * H, :])
        g = jnp.tanh(gates[2 * H:3 * H, :])
        to = jnp.tanh(gates[3 * H:4 * H, :])

        cth = ct.astype(jnp.bfloat16)
        c_new = 0.5 * ((cth + tf * cth) + (g + ti * g))
        tc = jnp.tanh(c_new)
        h_new = 0.5 * (tc + to * tc)

        ht_out_ref[H * p:H * p + H, :] = jnp.where(
            m, h_new.astype(jnp.float32), ht)
        ct_out_ref[H * p:H * p + H, :] = jnp.where(
            m, c_new.astype(jnp.float32), ct)


def kernel(corr_index, rela_ht, rela_ct, nei_index, W_emb, b_emb, W_ih, W_hh, b_ih, b_hh):
    ht = rela_ht.transpose(0, 2, 1).reshape(P * H, P)
    ct = rela_ct.transpose(0, 2, 1).reshape(P * H, P)
    corr_t = corr_index.transpose(0, 2, 1)
    ones_row = jnp.ones((P, 1, P), jnp.float32)
    corr_aug = jnp.concatenate([corr_t, ones_row], axis=1)
    corr_aug = corr_aug.reshape(P * 3, P).astype(jnp.bfloat16)
    nei = nei_index.astype(jnp.int32)

    w_emb_aug = jnp.concatenate([W_emb, b_emb[:, None]], axis=1)
    w_emb_aug = jnp.concatenate(
        [w_emb_aug, jnp.array([[0.0, 0.0, 1.0]], jnp.float32),
         jnp.zeros((EP - E - 1, 3), jnp.float32)], axis=0)

    w_emb_bd = jax.scipy.linalg.block_diag(
        *([w_emb_aug] * BP)).astype(jnp.bfloat16)

    bias = (b_ih + b_hh)[:, None]
    w_ih_aug = jnp.concatenate(
        [W_ih, bias, jnp.zeros((4 * H, EP - E - 1), jnp.float32)],
        axis=1)
    scale = jnp.where((jnp.arange(4 * H) < 2 * H) | (jnp.arange(4 * H) >= 3 * H),
                      0.5, 1.0)[:, None]
    w_cat = (jnp.concatenate([w_ih_aug, W_hh], axis=1)
             * scale).astype(jnp.bfloat16)

    grid = (P // BP,)
    ht_out, ct_out = pl.pallas_call(
        _lstm_block,
        grid=grid,
        in_specs=[
            pl.BlockSpec((3 * BP, P), lambda i: (i, 0)),
            pl.BlockSpec((H * BP, P), lambda i: (i, 0)),
            pl.BlockSpec((H * BP, P), lambda i: (i, 0)),
            pl.BlockSpec((BP, P), lambda i: (i, 0)),
            pl.BlockSpec((BP * EP, BP * 3), lambda i: (0, 0)),
            pl.BlockSpec((4 * H, KS), lambda i: (0, 0)),
        ],
        out_specs=[
            pl.BlockSpec((H * BP, P), lambda i: (i, 0)),
            pl.BlockSpec((H * BP, P), lambda i: (i, 0)),
        ],
        out_shape=[
            jax.ShapeDtypeStruct((P * H, P), jnp.float32),
            jax.ShapeDtypeStruct((P * H, P), jnp.float32),
        ],
        scratch_shapes=[pltpu.VMEM((BP * KS, P), jnp.bfloat16)],
        compiler_params=pltpu.CompilerParams(
            dimension_semantics=("arbitrary",),
        ),
    )(corr_aug, ht, ct, nei, w_emb_bd, w_cat)

    return (ht_out.reshape(P, H, P).transpose(0, 2, 1),
            ct_out.reshape(P, H, P).transpose(0, 2, 1))
